# baseline (speedup 1.0000x reference)
import functools

import jax
import jax.numpy as jnp
import numpy as np
from jax import lax
from jax.experimental import pallas as pl
from jax.experimental.pallas import tpu as pltpu
from jax.experimental.pallas import tpu_sc as plsc

_FIELDS = 26
_BATCH = 16384
_TABLE = 2600000

_NC, _NS = 2, 16
_FPC = _FIELDS // _NC
_HALF = _TABLE // _NC
_BPW = _BATCH // _NS
_CHUNK = _FPC * _BPW
_PRE = 5

_SPM = 1300096
_W1 = 1299968
_STG = 81280

_OFFS = np.where(
    np.arange(_FIELDS) < _FPC,
    np.arange(_FIELDS) * 100000,
    (np.arange(_FIELDS) - _FPC) * 100000 + (_HALF - _W1),
).astype(np.int32)

_SIDE_OFF = 2599808
_SIDE_DST = _SIDE_OFF - _W1
_SIDE_LEN = 256


def _sc_call(idx, table_row, side):
    mesh = plsc.VectorSubcoreMesh(core_axis_name="c", subcore_axis_name="s")

    @functools.partial(
        pl.kernel,
        out_type=jax.ShapeDtypeStruct((_NC, _BATCH), jnp.float32),
        mesh=mesh,
        scratch_types=[
            pltpu.VMEM_SHARED((_SPM,), jnp.float32),
            pltpu.VMEM((_CHUNK,), jnp.int32),
            pltpu.VMEM((_CHUNK,), jnp.float32),
            pltpu.VMEM((_BPW,), jnp.float32),
            pltpu.SemaphoreType.DMA,
            pltpu.SemaphoreType.DMA,
        ],
    )
    def k(idx_hbm, tab_hbm, side_hbm, out_hbm, shared, idx_v, vals_v, acc_v,
          sem, sem_pre):
        cid = lax.axis_index("c")
        sid = lax.axis_index("s")
        tab_flat = tab_hbm.at[0]

        hbm_win = tab_flat.at[pl.ds(cid * _W1, _HALF + (_HALF - _W1))]

        for j in range(_FPC):
            pltpu.async_copy(
                idx_hbm.at[cid * _FPC + j].at[pl.ds(sid * _BPW, _BPW)],
                idx_v.at[pl.ds(j * _BPW, _BPW)],
                sem_pre if j < _PRE else sem,
            )

        pltpu.make_async_copy(
            idx_hbm.at[0].at[pl.ds(0, _PRE * _BPW)],
            idx_v.at[pl.ds(0, _PRE * _BPW)],
            sem_pre,
        ).wait()
        for j in range(_PRE):
            pltpu.async_copy(
                hbm_win.at[idx_v.at[pl.ds(j * _BPW, _BPW)]],
                vals_v.at[pl.ds(j * _BPW, _BPW)],
                sem_pre,
            )

        src0 = cid * _W1 + sid * _STG

        @pl.when(sid < _NS - 1)
        def _stage_main():
            pltpu.sync_copy(
                tab_flat.at[pl.ds(src0, _STG)],
                shared.at[pl.ds(sid * _STG, _STG)],
            )

        @pl.when(jnp.logical_and(sid == _NS - 1, cid == 0))
        def _stage15_c0():
            pltpu.sync_copy(
                tab_flat.at[pl.ds(15 * _STG, _SPM - 15 * _STG)],
                shared.at[pl.ds(15 * _STG, _SPM - 15 * _STG)],
            )

        @pl.when(jnp.logical_and(sid == _NS - 1, cid == 1))
        def _stage15_c1():
            pltpu.sync_copy(
                tab_flat.at[pl.ds(_W1 + 15 * _STG, _W1 - 15 * _STG)],
                shared.at[pl.ds(15 * _STG, _W1 - 15 * _STG)],
            )

        @pl.when(jnp.logical_and(sid == 0, cid == 1))
        def _stage_tail():
            pltpu.sync_copy(side_hbm, shared.at[pl.ds(_SIDE_DST, _SIDE_LEN)])

        pltpu.make_async_copy(
            idx_hbm.at[0].at[pl.ds(_PRE * _BPW, _CHUNK - _PRE * _BPW)],
            idx_v.at[pl.ds(_PRE * _BPW, _CHUNK - _PRE * _BPW)],
            sem,
        ).wait()

        plsc.subcore_barrier()

        for j in range(_PRE, _FPC):
            pltpu.async_copy(
                shared.at[idx_v.at[pl.ds(j * _BPW, _BPW)]],
                vals_v.at[pl.ds(j * _BPW, _BPW)],
                sem,
            )
        pltpu.make_async_copy(
            shared.at[idx_v.at[pl.ds(_PRE * _BPW, _CHUNK - _PRE * _BPW)]],
            vals_v.at[pl.ds(_PRE * _BPW, _CHUNK - _PRE * _BPW)],
            sem,
        ).wait()
        pltpu.make_async_copy(
            hbm_win.at[idx_v.at[pl.ds(0, _PRE * _BPW)]],
            vals_v.at[pl.ds(0, _PRE * _BPW)],
            sem_pre,
        ).wait()

        def reduce_col(col16, _):
            col = col16 * 16
            acc = vals_v[pl.ds(col, 16)]
            for j in range(1, _FPC):
                acc = acc + vals_v[pl.ds(j * _BPW + col, 16)]
            acc_v[pl.ds(col, 16)] = acc
            return 0
        lax.fori_loop(0, _BPW // 16, reduce_col, 0)

        pltpu.sync_copy(acc_v, out_hbm.at[cid].at[pl.ds(sid * _BPW, _BPW)])

    return k(idx, table_row, side)


def kernel(x, table, bias):
    offs = jnp.asarray(_OFFS, dtype=x.dtype)
    idx = (x + offs).astype(jnp.int32).T
    table_row = jnp.transpose(table, (1, 0))
    side = jnp.concatenate(
        [table_row[0, _SIDE_OFF:_TABLE],
         jnp.zeros((_SIDE_LEN - (_TABLE - _SIDE_OFF),), jnp.float32)]
    )
    partials = _sc_call(idx, table_row, side)
    return partials[0] + partials[1] + bias.astype(jnp.float32)[0]


# speedup vs baseline: 1.0234x; 1.0234x over previous
import functools

import jax
import jax.numpy as jnp
import numpy as np
from jax import lax
from jax.experimental import pallas as pl
from jax.experimental.pallas import tpu as pltpu
from jax.experimental.pallas import tpu_sc as plsc

_FIELDS = 26
_BATCH = 16384
_TABLE = 2600000

_NC, _NS = 2, 16
_FPC = _FIELDS // _NC
_HALF = _TABLE // _NC
_BPW = _BATCH // _NS
_CHUNK = _FPC * _BPW
_PRE = 4

_SPM = 1300096
_W1 = 1299968
_STG = 81280

_OFFS = np.where(
    np.arange(_FIELDS) < _FPC,
    np.arange(_FIELDS) * 100000,
    (np.arange(_FIELDS) - _FPC) * 100000 + (_HALF - _W1),
).astype(np.int32)

_SIDE_OFF = 2599808
_SIDE_DST = _SIDE_OFF - _W1
_SIDE_LEN = _TABLE - _SIDE_OFF


def _sc_call(idx, table_row):
    mesh = plsc.VectorSubcoreMesh(core_axis_name="c", subcore_axis_name="s")

    @functools.partial(
        pl.kernel,
        out_type=jax.ShapeDtypeStruct((_NC, _BATCH), jnp.float32),
        mesh=mesh,
        scratch_types=[
            pltpu.VMEM_SHARED((_SPM,), jnp.float32),
            pltpu.VMEM((_CHUNK,), jnp.int32),
            pltpu.VMEM((_CHUNK,), jnp.float32),
            pltpu.VMEM((_BPW,), jnp.float32),
            pltpu.VMEM((_SIDE_LEN,), jnp.float32),
            pltpu.SemaphoreType.DMA,
            pltpu.SemaphoreType.DMA,
        ],
    )
    def k(idx_hbm, tab_hbm, out_hbm, shared, idx_v, vals_v, acc_v, tail_v,
          sem, sem_pre):
        cid = lax.axis_index("c")
        sid = lax.axis_index("s")
        tab_flat = tab_hbm.at[0]

        hbm_win = tab_flat.at[pl.ds(cid * _W1, _HALF + (_HALF - _W1))]

        for j in range(_FPC):
            pltpu.async_copy(
                idx_hbm.at[cid * _FPC + j].at[pl.ds(sid * _BPW, _BPW)],
                idx_v.at[pl.ds(j * _BPW, _BPW)],
                sem_pre if j < _PRE else sem,
            )

        pltpu.make_async_copy(
            idx_hbm.at[0].at[pl.ds(0, _PRE * _BPW)],
            idx_v.at[pl.ds(0, _PRE * _BPW)],
            sem_pre,
        ).wait()
        for j in range(_PRE):
            pltpu.async_copy(
                hbm_win.at[idx_v.at[pl.ds(j * _BPW, _BPW)]],
                vals_v.at[pl.ds(j * _BPW, _BPW)],
                sem_pre,
            )

        src0 = cid * _W1 + sid * _STG

        @pl.when(sid < _NS - 1)
        def _stage_main():
            pltpu.sync_copy(
                tab_flat.at[pl.ds(src0, _STG)],
                shared.at[pl.ds(sid * _STG, _STG)],
            )

        @pl.when(jnp.logical_and(sid == _NS - 1, cid == 0))
        def _stage15_c0():
            pltpu.sync_copy(
                tab_flat.at[pl.ds(15 * _STG, _SPM - 15 * _STG)],
                shared.at[pl.ds(15 * _STG, _SPM - 15 * _STG)],
            )

        @pl.when(jnp.logical_and(sid == _NS - 1, cid == 1))
        def _stage15_c1():
            pltpu.sync_copy(
                tab_flat.at[pl.ds(_W1 + 15 * _STG, _W1 - 15 * _STG)],
                shared.at[pl.ds(15 * _STG, _W1 - 15 * _STG)],
            )

        @pl.when(jnp.logical_and(sid == 0, cid == 1))
        def _stage_tail():
            pltpu.sync_copy(tab_flat.at[pl.ds(_SIDE_OFF, _SIDE_LEN)], tail_v)
            pltpu.sync_copy(tail_v, shared.at[pl.ds(_SIDE_DST, _SIDE_LEN)])

        pltpu.make_async_copy(
            idx_hbm.at[0].at[pl.ds(_PRE * _BPW, _CHUNK - _PRE * _BPW)],
            idx_v.at[pl.ds(_PRE * _BPW, _CHUNK - _PRE * _BPW)],
            sem,
        ).wait()

        plsc.subcore_barrier()

        for j in range(_PRE, _FPC):
            pltpu.async_copy(
                shared.at[idx_v.at[pl.ds(j * _BPW, _BPW)]],
                vals_v.at[pl.ds(j * _BPW, _BPW)],
                sem,
            )
        pltpu.make_async_copy(
            shared.at[idx_v.at[pl.ds(_PRE * _BPW, _CHUNK - _PRE * _BPW)]],
            vals_v.at[pl.ds(_PRE * _BPW, _CHUNK - _PRE * _BPW)],
            sem,
        ).wait()
        pltpu.make_async_copy(
            hbm_win.at[idx_v.at[pl.ds(0, _PRE * _BPW)]],
            vals_v.at[pl.ds(0, _PRE * _BPW)],
            sem_pre,
        ).wait()

        def reduce_col(col16, _):
            col = col16 * 16
            acc = vals_v[pl.ds(col, 16)]
            for j in range(1, _FPC):
                acc = acc + vals_v[pl.ds(j * _BPW + col, 16)]
            acc_v[pl.ds(col, 16)] = acc
            return 0
        lax.fori_loop(0, _BPW // 16, reduce_col, 0)

        pltpu.sync_copy(acc_v, out_hbm.at[cid].at[pl.ds(sid * _BPW, _BPW)])

    return k(idx, table_row)


def kernel(x, table, bias):
    offs = jnp.asarray(_OFFS, dtype=x.dtype)
    idx = (x + offs).astype(jnp.int32).T
    table_row = jnp.transpose(table, (1, 0))
    partials = _sc_call(idx, table_row)
    return partials[0] + partials[1] + bias.astype(jnp.float32)[0]
